# baseline (speedup 1.0000x reference)
import functools

import jax
import jax.numpy as jnp
from jax import lax
from jax.experimental import pallas as pl
from jax.experimental.pallas import tpu as pltpu
from jax.experimental.pallas import tpu_sc as plsc

_NUM_CORES = 2
_NUM_SUBCORES = 16
_NUM_WORKERS = _NUM_CORES * _NUM_SUBCORES
_LANES = 16
_BLK = 256


def _sc_broadcast(rows, seq_len, dim, embeddings):
    full_steps = rows // _NUM_WORKERS
    rem_rows = rows - full_steps * _NUM_WORKERS
    writes_per_row = seq_len // _BLK
    rem_chunk = seq_len // _NUM_WORKERS
    n_staged = full_steps + rem_rows
    vchunks = dim // _LANES

    mesh = plsc.VectorSubcoreMesh(core_axis_name="c", subcore_axis_name="s")

    @functools.partial(
        pl.kernel,
        out_type=jax.ShapeDtypeStruct((rows, seq_len, dim), jnp.float32),
        mesh=mesh,
        scratch_types=[
            pltpu.VMEM((n_staged, dim), jnp.float32),
            pltpu.VMEM((_BLK, dim), jnp.float32),
            pltpu.VMEM((_BLK, dim), jnp.float32),
        ]
        + [pltpu.SemaphoreType.DMA] * (n_staged + 2),
    )
    def kern(emb_hbm, out_hbm, myrows, buf0, buf1, *sems):
        wid = lax.axis_index("s") * _NUM_CORES + lax.axis_index("c")
        buf = (buf0, buf1)
        ssem = sems[:n_staged]
        wsem = sems[n_staged:]

        stage = [
            pltpu.async_copy(
                emb_hbm.at[pl.ds(step * _NUM_WORKERS + wid, 1), :],
                myrows.at[pl.ds(step, 1), :],
                ssem[step],
            )
            for step in range(full_steps)
        ]
        stage += [
            pltpu.async_copy(
                emb_hbm.at[pl.ds(full_steps * _NUM_WORKERS + r, 1), :],
                myrows.at[pl.ds(full_steps + r, 1), :],
                ssem[full_steps + r],
            )
            for r in range(rem_rows)
        ]
        staged = [False] * n_staged

        def build(slot, step, ncopies):
            if not staged[step]:
                stage[step].wait()
                staged[step] = True
            regs = [myrows[step, pl.ds(v * _LANES, _LANES)] for v in range(vchunks)]

            def body(r, c):
                for v in range(vchunks):
                    buf[slot][r, pl.ds(v * _LANES, _LANES)] = regs[v]
                return c

            lax.fori_loop(0, ncopies, body, 0)

        def fire_writes(slot, row):
            return [
                pltpu.async_copy(
                    buf[slot],
                    out_hbm.at[row, pl.ds(j * _BLK, _BLK), :],
                    wsem[slot],
                )
                for j in range(writes_per_row)
            ]

        pend_w = {0: [], 1: []}

        first = 64 if (full_steps and seq_len % 64 == 0) else _BLK
        if full_steps:
            build(0, 0, first)
            pend_w[0] = [
                pltpu.async_copy(
                    buf[0].at[pl.ds(0, first), :],
                    out_hbm.at[wid, pl.ds(j * first, first), :],
                    wsem[0],
                )
                for j in range(seq_len // first)
            ]
        for step in range(full_steps):
            slot = step % 2
            row = step * _NUM_WORKERS + wid
            if step > 0:
                pend_w[slot] = fire_writes(slot, row)
            for cp in pend_w[1 - slot]:
                cp.wait()
            pend_w[1 - slot] = []
            if step + 1 < full_steps:
                build(1 - slot, step + 1, _BLK)
            elif rem_rows:
                build(1 - slot, full_steps, rem_chunk)

        if not full_steps and rem_rows:
            build(0, 0, rem_chunk)
        for r in range(rem_rows):
            step = full_steps + r
            slot = step % 2
            row = full_steps * _NUM_WORKERS + r
            pend_w[slot] = [
                pltpu.async_copy(
                    buf[slot].at[pl.ds(0, rem_chunk), :],
                    out_hbm.at[row, pl.ds(wid * rem_chunk, rem_chunk), :],
                    wsem[slot],
                )
            ]
            for cp in pend_w[1 - slot]:
                cp.wait()
            pend_w[1 - slot] = []
            if r + 1 < rem_rows:
                build(1 - slot, step + 1, rem_chunk)

        for slot in (0, 1):
            for cp in pend_w[slot]:
                cp.wait()

    return kern(embeddings)


def kernel(time, embeddings):
    batch_size, seq_len = time.shape
    rows, dim = embeddings.shape
    return _sc_broadcast(rows, seq_len, dim, embeddings)


# speedup vs baseline: 1.0007x; 1.0007x over previous
import functools

import jax
import jax.numpy as jnp
from jax import lax
from jax.experimental import pallas as pl
from jax.experimental.pallas import tpu as pltpu
from jax.experimental.pallas import tpu_sc as plsc

_NUM_CORES = 2
_NUM_SUBCORES = 16
_NUM_WORKERS = _NUM_CORES * _NUM_SUBCORES
_LANES = 16
_BLK = 256


def _sc_broadcast(rows, seq_len, dim, embeddings):
    full_steps = rows // _NUM_WORKERS
    rem_rows = rows - full_steps * _NUM_WORKERS
    writes_per_row = seq_len // _BLK
    rem_chunk = seq_len // _NUM_WORKERS
    n_staged = full_steps + rem_rows
    vchunks = dim // _LANES

    mesh = plsc.VectorSubcoreMesh(core_axis_name="c", subcore_axis_name="s")

    @functools.partial(
        pl.kernel,
        out_type=jax.ShapeDtypeStruct((rows, seq_len, dim), jnp.float32),
        mesh=mesh,
        scratch_types=[
            pltpu.VMEM((n_staged, dim), jnp.float32),
            pltpu.VMEM((_BLK, dim), jnp.float32),
            pltpu.VMEM((_BLK, dim), jnp.float32),
        ]
        + [pltpu.SemaphoreType.DMA] * (n_staged + 2),
    )
    def kern(emb_hbm, out_hbm, myrows, buf0, buf1, *sems):
        wid = lax.axis_index("s") * _NUM_CORES + lax.axis_index("c")
        buf = (buf0, buf1)
        ssem = sems[:n_staged]
        wsem = sems[n_staged:]

        stage = [
            pltpu.async_copy(
                emb_hbm.at[pl.ds(step * _NUM_WORKERS + wid, 1), :],
                myrows.at[pl.ds(step, 1), :],
                ssem[step],
            )
            for step in range(full_steps)
        ]
        stage += [
            pltpu.async_copy(
                emb_hbm.at[pl.ds(full_steps * _NUM_WORKERS + r, 1), :],
                myrows.at[pl.ds(full_steps + r, 1), :],
                ssem[full_steps + r],
            )
            for r in range(rem_rows)
        ]
        staged = [False] * n_staged

        def build(slot, step, ncopies):
            if not staged[step]:
                stage[step].wait()
                staged[step] = True
            regs = [myrows[step, pl.ds(v * _LANES, _LANES)] for v in range(vchunks)]

            def body(r, c):
                for v in range(vchunks):
                    buf[slot][r, pl.ds(v * _LANES, _LANES)] = regs[v]
                return c

            lax.fori_loop(0, ncopies, body, 0)

        def fire_writes(slot, row):
            return [
                pltpu.async_copy(
                    buf[slot],
                    out_hbm.at[row, pl.ds(j * _BLK, _BLK), :],
                    wsem[slot],
                )
                for j in range(writes_per_row)
            ]

        pend_w = {0: [], 1: []}

        if full_steps:
            build(0, 0, _BLK)
        for step in range(full_steps):
            slot = step % 2
            row = step * _NUM_WORKERS + wid
            pend_w[slot] = fire_writes(slot, row)
            for cp in pend_w[1 - slot]:
                cp.wait()
            pend_w[1 - slot] = []
            if step + 1 < full_steps:
                build(1 - slot, step + 1, _BLK)
            elif rem_rows:
                build(1 - slot, full_steps, rem_chunk)

        if not full_steps and rem_rows:
            build(0, 0, rem_chunk)
        for r in range(rem_rows):
            step = full_steps + r
            slot = step % 2
            row = full_steps * _NUM_WORKERS + r
            pend_w[slot] = [
                pltpu.async_copy(
                    buf[slot].at[pl.ds(0, rem_chunk), :],
                    out_hbm.at[row, pl.ds(wid * rem_chunk, rem_chunk), :],
                    wsem[slot],
                )
            ]
            for cp in pend_w[1 - slot]:
                cp.wait()
            pend_w[1 - slot] = []
            if r + 1 < rem_rows:
                build(1 - slot, step + 1, rem_chunk)

        for slot in (0, 1):
            for cp in pend_w[slot]:
                cp.wait()

    return kern(embeddings)


def kernel(time, embeddings):
    batch_size, seq_len = time.shape
    rows, dim = embeddings.shape
    return _sc_broadcast(rows, seq_len, dim, embeddings)
